# baseline (speedup 1.0000x reference)
import jax
import jax.numpy as jnp
from jax.experimental import pallas as pl

NUM_TYPES = 128
N_NODES = 100000
BLOCK = 2000


def _onehot_body(types_ref, out0_ref, out1_ref):
    t = types_ref[...]
    cols = jax.lax.broadcasted_iota(jnp.int32, (BLOCK, NUM_TYPES), 1)
    oh = (cols == t).astype(jnp.float32)
    out0_ref[...] = oh
    out1_ref[...] = oh


def kernel(node_types, pos):
    grid = (N_NODES // BLOCK,)
    out = pl.pallas_call(
        _onehot_body,
        grid=grid,
        in_specs=[pl.BlockSpec((BLOCK, 1), lambda i: (i, 0))],
        out_specs=[
            pl.BlockSpec((BLOCK, NUM_TYPES), lambda i: (i, 0)),
            pl.BlockSpec((BLOCK, NUM_TYPES), lambda i: (i, 0)),
        ],
        out_shape=[
            jax.ShapeDtypeStruct((N_NODES, NUM_TYPES), pos.dtype),
            jax.ShapeDtypeStruct((N_NODES, NUM_TYPES), pos.dtype),
        ],
    )(node_types)
    return (out[0], out[1])


# speedup vs baseline: 1.2039x; 1.2039x over previous
import jax
import jax.numpy as jnp
from jax.experimental import pallas as pl

NUM_TYPES = 128
N_NODES = 100000
BLOCK = 5000


def _onehot_body(types_ref, out0_ref, out1_ref):
    t = types_ref[...]
    cols = jax.lax.broadcasted_iota(jnp.int32, (BLOCK, NUM_TYPES), 1)
    oh = (cols == t).astype(jnp.float32)
    out0_ref[...] = oh
    out1_ref[...] = oh


def kernel(node_types, pos):
    grid = (N_NODES // BLOCK,)
    out = pl.pallas_call(
        _onehot_body,
        grid=grid,
        in_specs=[pl.BlockSpec((BLOCK, 1), lambda i: (i, 0))],
        out_specs=[
            pl.BlockSpec((BLOCK, NUM_TYPES), lambda i: (i, 0)),
            pl.BlockSpec((BLOCK, NUM_TYPES), lambda i: (i, 0)),
        ],
        out_shape=[
            jax.ShapeDtypeStruct((N_NODES, NUM_TYPES), pos.dtype),
            jax.ShapeDtypeStruct((N_NODES, NUM_TYPES), pos.dtype),
        ],
    )(node_types)
    return (out[0], out[1])


# speedup vs baseline: 1.2522x; 1.0401x over previous
import jax
import jax.numpy as jnp
from jax.experimental import pallas as pl

NUM_TYPES = 128
N_NODES = 100000
BLOCK = 10000


def _onehot_body(types_ref, out0_ref, out1_ref):
    t = types_ref[...]
    cols = jax.lax.broadcasted_iota(jnp.int32, (BLOCK, NUM_TYPES), 1)
    oh = (cols == t).astype(jnp.float32)
    out0_ref[...] = oh
    out1_ref[...] = oh


def kernel(node_types, pos):
    grid = (N_NODES // BLOCK,)
    out = pl.pallas_call(
        _onehot_body,
        grid=grid,
        in_specs=[pl.BlockSpec((BLOCK, 1), lambda i: (i, 0))],
        out_specs=[
            pl.BlockSpec((BLOCK, NUM_TYPES), lambda i: (i, 0)),
            pl.BlockSpec((BLOCK, NUM_TYPES), lambda i: (i, 0)),
        ],
        out_shape=[
            jax.ShapeDtypeStruct((N_NODES, NUM_TYPES), pos.dtype),
            jax.ShapeDtypeStruct((N_NODES, NUM_TYPES), pos.dtype),
        ],
    )(node_types)
    return (out[0], out[1])


# speedup vs baseline: 1.2721x; 1.0159x over previous
import jax
import jax.numpy as jnp
from jax.experimental import pallas as pl

NUM_TYPES = 128
N_NODES = 100000
BLOCK = 20000


def _onehot_body(types_ref, out0_ref, out1_ref):
    t = types_ref[...]
    cols = jax.lax.broadcasted_iota(jnp.int32, (BLOCK, NUM_TYPES), 1)
    oh = (cols == t).astype(jnp.float32)
    out0_ref[...] = oh
    out1_ref[...] = oh


def kernel(node_types, pos):
    grid = (N_NODES // BLOCK,)
    out = pl.pallas_call(
        _onehot_body,
        grid=grid,
        in_specs=[pl.BlockSpec((BLOCK, 1), lambda i: (i, 0))],
        out_specs=[
            pl.BlockSpec((BLOCK, NUM_TYPES), lambda i: (i, 0)),
            pl.BlockSpec((BLOCK, NUM_TYPES), lambda i: (i, 0)),
        ],
        out_shape=[
            jax.ShapeDtypeStruct((N_NODES, NUM_TYPES), pos.dtype),
            jax.ShapeDtypeStruct((N_NODES, NUM_TYPES), pos.dtype),
        ],
    )(node_types)
    return (out[0], out[1])


# speedup vs baseline: 1.3741x; 1.0802x over previous
import jax
import jax.numpy as jnp
from jax import lax
from jax.experimental import pallas as pl
from jax.experimental.pallas import tpu as pltpu
from jax.experimental.pallas import tpu_sc as plsc

NUM_TYPES = 128
N_NODES = 100000

_SPAN = 3200
_CHUNK = 800
_NCHUNK = _SPAN // _CHUNK
_GROUPS = _CHUNK // 16

def _sc_body(types_hbm, zeros_hbm, out0_hbm, out1_hbm, types_v, buf, sem):
    wid = lax.axis_index("s") * 2 + lax.axis_index("c")
    base = jnp.minimum(wid * _SPAN, N_NODES - _SPAN)
    pltpu.sync_copy(types_hbm.at[pl.ds(base, _SPAN)], types_v)
    pltpu.sync_copy(zeros_hbm, buf)
    ones16 = jnp.ones((16,), jnp.float32)
    zeros16 = jnp.zeros((16,), jnp.float32)
    iota16 = lax.iota(jnp.int32, 16)

    def do_chunk(c, _):
        def scat(g, _):
            t = types_v[pl.ds(c * _CHUNK + g * 16, 16)]
            plsc.store_scatter(buf, [(g * 16 + iota16) * NUM_TYPES + t], ones16)
            return 0

        lax.fori_loop(0, _GROUPS, scat, 0)
        flat0 = (base + c * _CHUNK) * NUM_TYPES
        cp0 = pltpu.async_copy(buf, out0_hbm.at[pl.ds(flat0, _CHUNK * NUM_TYPES)], sem)
        cp1 = pltpu.async_copy(buf, out1_hbm.at[pl.ds(flat0, _CHUNK * NUM_TYPES)], sem)
        cp0.wait()
        cp1.wait()

        def unscat(g, _):
            t = types_v[pl.ds(c * _CHUNK + g * 16, 16)]
            plsc.store_scatter(buf, [(g * 16 + iota16) * NUM_TYPES + t], zeros16)
            return 0

        lax.fori_loop(0, _GROUPS, unscat, 0)
        return 0

    lax.fori_loop(0, _NCHUNK, do_chunk, 0)


def kernel(node_types, pos):
    types = jnp.reshape(node_types, (N_NODES,))
    zeros = jnp.zeros((_CHUNK * NUM_TYPES,), jnp.float32)
    mesh = plsc.VectorSubcoreMesh(core_axis_name="c", subcore_axis_name="s")
    k = pl.kernel(
        _sc_body,
        out_type=[
            jax.ShapeDtypeStruct((N_NODES * NUM_TYPES,), jnp.float32),
            jax.ShapeDtypeStruct((N_NODES * NUM_TYPES,), jnp.float32),
        ],
        mesh=mesh,
        compiler_params=pltpu.CompilerParams(needs_layout_passes=False),
        scratch_types=[
            pltpu.VMEM((_SPAN,), jnp.int32),
            pltpu.VMEM((_CHUNK * NUM_TYPES,), jnp.float32),
            pltpu.SemaphoreType.DMA,
        ],
    )
    out0, out1 = k(types, zeros)
    shape = (N_NODES, NUM_TYPES)
    return (jnp.reshape(out0, shape), jnp.reshape(out1, shape))
